# baseline (speedup 1.0000x reference)
import functools

import jax
import jax.numpy as jnp
from jax import lax
from jax.experimental import pallas as pl
from jax.experimental.pallas import tpu as pltpu
from jax.experimental.pallas import tpu_sc as plsc

NUM_SRC = 65536
NUM_DST = 65536
BATCH = 64

NC = 2
NS = 16
CB = 16
NCHUNK = BATCH // CB
PASSES = NCHUNK // NC
BLK = 128
K = 16
EBLK = K * BLK
NBUF = 2
ZR = 256
TROWS = NUM_DST // NS


def _splat(vec16, u):
    idx = jnp.full((16, 1), u, jnp.int32)
    dnums = lax.GatherDimensionNumbers(
        offset_dims=(), collapsed_slice_dims=(0,), start_index_map=(0,))
    return lax.gather(vec16, idx, dnums, (1,),
                      mode=lax.GatherScatterMode.PROMISE_IN_BOUNDS)


def _body(nblocks, xg, colr, rowr, valr, out,
          acc, colb, rowb, valb, rbuf, zbuf, sem_i, sem_g, sem_s):
    cid = lax.axis_index("c")
    sid = lax.axis_index("s")

    def zfill(i, _):
        zbuf[i, :] = jnp.zeros((CB,), jnp.float32)
        return 0
    lax.fori_loop(0, ZR, zfill, 0)

    for p in range(PASSES):
        chunk = cid * PASSES + p

        mrows = nblocks * NS * K

        def fire_idx(b, buf):
            boff = (sid * nblocks + b) * K
            pltpu.async_copy(
                colr.at[pl.ds(chunk * mrows + boff, K)], colb.at[buf], sem_i)
            pltpu.async_copy(rowr.at[pl.ds(boff, K)], rowb.at[buf], sem_i)
            pltpu.async_copy(valr.at[pl.ds(boff, K)], valb.at[buf], sem_i)

        def wait_idx(b, buf):
            boff = (sid * nblocks + b) * K
            pltpu.make_async_copy(
                colr.at[pl.ds(chunk * mrows + boff, K)], colb.at[buf],
                sem_i).wait()
            pltpu.make_async_copy(
                rowr.at[pl.ds(boff, K)], rowb.at[buf], sem_i).wait()
            pltpu.make_async_copy(
                valr.at[pl.ds(boff, K)], valb.at[buf], sem_i).wait()

        def fire_gathers(buf):
            for j in range(K):
                pltpu.async_copy(
                    xg.at[colb.at[buf].at[j]], rbuf.at[j], sem_g)

        def wait_gathers(buf):
            for j in range(K):
                pltpu.make_async_copy(
                    xg.at[colb.at[buf].at[j]], rbuf.at[j],
                    sem_g).wait()

        def fire_scatters(buf):
            for j in range(K):
                pltpu.async_copy(
                    rbuf.at[j], acc.at[rowb.at[buf].at[j]], sem_s,
                    add=True)

        def wait_scatters(buf):
            for j in range(K):
                pltpu.make_async_copy(
                    rbuf.at[j], acc.at[rowb.at[buf].at[j]],
                    sem_s).wait()

        def mult_j(buf, j):
            def tbody(t, _):
                base = t * 16
                vals16 = valb[buf, j, pl.ds(base, 16)]
                for u in range(16):
                    sp = _splat(vals16, u)
                    rbuf[j, base + u, :] = (
                        rbuf[j, base + u, :] * sp)
                return 0
            lax.fori_loop(0, BLK // 16, tbody, 0)

        for z in range(TROWS // ZR):
            pltpu.sync_copy(zbuf, acc.at[pl.ds(sid * TROWS + z * ZR, ZR)])
        plsc.subcore_barrier()

        fire_idx(0, 0)

        def blk_step(q, _):
            for r in range(NBUF):
                b = NBUF * q + r
                s_cur = r
                s_nxt = (r + 1) % NBUF
                wait_idx(b, s_cur)

                @pl.when(b >= 1)
                def _():
                    wait_scatters(s_nxt)

                @pl.when(b + 1 < nblocks)
                def _():
                    fire_idx(b + 1, s_nxt)
                fire_gathers(s_cur)
                for j in range(K):
                    pltpu.make_async_copy(
                        xg.at[colb.at[s_cur].at[j]], rbuf.at[j],
                        sem_g).wait()
                    mult_j(s_cur, j)
                fire_scatters(s_cur)
            return 0

        lax.fori_loop(0, nblocks // NBUF, blk_step, 0)
        wait_scatters((nblocks - 1) % NBUF)
        plsc.subcore_barrier()

        pltpu.sync_copy(
            acc.at[pl.ds(sid * TROWS, TROWS)],
            out.at[pl.ds(chunk * NUM_DST + sid * TROWS, TROWS)])


def kernel(x, row, col, values):
    e = row.shape[0]
    eb = NBUF * NS * EBLK
    e_pad = ((e + eb - 1) // eb) * eb
    pad = e_pad - e
    nblocks = e_pad // (NS * EBLK)

    colp = jnp.pad(col, (0, pad))
    shift = (jnp.arange(NCHUNK, dtype=jnp.int32) * NUM_SRC)[:, None]
    colr = (colp[None, :] + shift).reshape(NCHUNK * e_pad // BLK, BLK)
    rowr = jnp.pad(row, (0, pad)).reshape(e_pad // BLK, BLK)
    valr = jnp.pad(values, (0, pad)).reshape(e_pad // BLK, BLK)
    xg = (x.reshape(NCHUNK, CB, NUM_SRC)
          .transpose(0, 2, 1)
          .reshape(NCHUNK * NUM_SRC, CB))

    mesh = plsc.VectorSubcoreMesh(core_axis_name="c", subcore_axis_name="s")
    f = pl.kernel(
        functools.partial(_body, nblocks),
        out_type=jax.ShapeDtypeStruct((NCHUNK * NUM_DST, CB), jnp.float32),
        mesh=mesh,
        compiler_params=pltpu.CompilerParams(use_tc_tiling_on_sc=False),
        scratch_types=[
            pltpu.VMEM_SHARED((NUM_DST, CB), jnp.float32),
            pltpu.VMEM((NBUF, K, BLK), jnp.int32),
            pltpu.VMEM((NBUF, K, BLK), jnp.int32),
            pltpu.VMEM((NBUF, K, BLK), jnp.float32),
            pltpu.VMEM((K, BLK, CB), jnp.float32),
            pltpu.VMEM((ZR, CB), jnp.float32),
            pltpu.SemaphoreType.DMA,
            pltpu.SemaphoreType.DMA,
            pltpu.SemaphoreType.DMA,
        ],
    )
    outg = f(xg, colr, rowr, valr)
    return (outg.reshape(NCHUNK, NUM_DST, CB)
            .transpose(0, 2, 1)
            .reshape(BATCH, NUM_DST))


# speedup vs baseline: 1.0459x; 1.0459x over previous
import functools

import jax
import jax.numpy as jnp
from jax import lax
from jax.experimental import pallas as pl
from jax.experimental.pallas import tpu as pltpu
from jax.experimental.pallas import tpu_sc as plsc

NUM_SRC = 65536
NUM_DST = 65536
BATCH = 64

NC = 2
NS = 16
CB = 16
NCHUNK = BATCH // CB
PASSES = NCHUNK // NC
BLK = 128
K = 16
EBLK = K * BLK
NBUF = 2
ZR = 256
TROWS = NUM_DST // NS


def _splat(vec16, u):
    idx = jnp.full((16, 1), u, jnp.int32)
    dnums = lax.GatherDimensionNumbers(
        offset_dims=(), collapsed_slice_dims=(0,), start_index_map=(0,))
    return lax.gather(vec16, idx, dnums, (1,),
                      mode=lax.GatherScatterMode.PROMISE_IN_BOUNDS)


def _body(nblocks, xg, colr, rowr, valr, out,
          acc, colb, rowb, valb, rbuf, zbuf, sem_i, sem_g, sem_s):
    cid = lax.axis_index("c")
    sid = lax.axis_index("s")

    def zfill(i, _):
        zbuf[i, :] = jnp.zeros((CB,), jnp.float32)
        return 0
    lax.fori_loop(0, ZR, zfill, 0)

    for p in range(PASSES):
        chunk = cid * PASSES + p

        xgc = xg.at[pl.ds(chunk * NUM_SRC, NUM_SRC)]

        def fire_idx(b, buf):
            boff = (sid * nblocks + b) * K
            pltpu.async_copy(colr.at[pl.ds(boff, K)], colb.at[buf], sem_i)
            pltpu.async_copy(rowr.at[pl.ds(boff, K)], rowb.at[buf], sem_i)
            pltpu.async_copy(valr.at[pl.ds(boff, K)], valb.at[buf], sem_i)

        def wait_idx(b, buf):
            boff = (sid * nblocks + b) * K
            pltpu.make_async_copy(
                colr.at[pl.ds(boff, K)], colb.at[buf], sem_i).wait()
            pltpu.make_async_copy(
                rowr.at[pl.ds(boff, K)], rowb.at[buf], sem_i).wait()
            pltpu.make_async_copy(
                valr.at[pl.ds(boff, K)], valb.at[buf], sem_i).wait()

        def fire_gathers(buf):
            for j in range(K):
                pltpu.async_copy(
                    xgc.at[colb.at[buf].at[j]], rbuf.at[j], sem_g)

        def wait_gathers(buf):
            for j in range(K):
                pltpu.make_async_copy(
                    xg.at[colb.at[buf].at[j]], rbuf.at[j],
                    sem_g).wait()

        def fire_scatters(buf):
            for j in range(K):
                pltpu.async_copy(
                    rbuf.at[j], acc.at[rowb.at[buf].at[j]], sem_s,
                    add=True)

        def wait_scatters(buf):
            for j in range(K):
                pltpu.make_async_copy(
                    rbuf.at[j], acc.at[rowb.at[buf].at[j]],
                    sem_s).wait()

        def mult_j(buf, j):
            def tbody(t, _):
                base = t * 16
                vals16 = valb[buf, j, pl.ds(base, 16)]
                for u in range(16):
                    sp = _splat(vals16, u)
                    rbuf[j, base + u, :] = (
                        rbuf[j, base + u, :] * sp)
                return 0
            lax.fori_loop(0, BLK // 16, tbody, 0)

        for z in range(TROWS // ZR):
            pltpu.sync_copy(zbuf, acc.at[pl.ds(sid * TROWS + z * ZR, ZR)])
        plsc.subcore_barrier()

        fire_idx(0, 0)

        def blk_step(q, _):
            for r in range(NBUF):
                b = NBUF * q + r
                s_cur = r
                s_nxt = (r + 1) % NBUF
                wait_idx(b, s_cur)

                @pl.when(b >= 1)
                def _():
                    wait_scatters(s_nxt)

                @pl.when(b + 1 < nblocks)
                def _():
                    fire_idx(b + 1, s_nxt)
                fire_gathers(s_cur)
                for j in range(K):
                    pltpu.make_async_copy(
                        xgc.at[colb.at[s_cur].at[j]], rbuf.at[j],
                        sem_g).wait()
                    mult_j(s_cur, j)
                fire_scatters(s_cur)
            return 0

        lax.fori_loop(0, nblocks // NBUF, blk_step, 0)
        wait_scatters((nblocks - 1) % NBUF)
        plsc.subcore_barrier()

        pltpu.sync_copy(
            acc.at[pl.ds(sid * TROWS, TROWS)],
            out.at[pl.ds(chunk * NUM_DST + sid * TROWS, TROWS)])


def kernel(x, row, col, values):
    e = row.shape[0]
    eb = NBUF * NS * EBLK
    e_pad = ((e + eb - 1) // eb) * eb
    pad = e_pad - e
    nblocks = e_pad // (NS * EBLK)

    colr = jnp.pad(col, (0, pad)).reshape(e_pad // BLK, BLK)
    rowr = jnp.pad(row, (0, pad)).reshape(e_pad // BLK, BLK)
    valr = jnp.pad(values, (0, pad)).reshape(e_pad // BLK, BLK)
    xg = (x.reshape(NCHUNK, CB, NUM_SRC)
          .transpose(0, 2, 1)
          .reshape(NCHUNK * NUM_SRC, CB))

    mesh = plsc.VectorSubcoreMesh(core_axis_name="c", subcore_axis_name="s")
    f = pl.kernel(
        functools.partial(_body, nblocks),
        out_type=jax.ShapeDtypeStruct((NCHUNK * NUM_DST, CB), jnp.float32),
        mesh=mesh,
        compiler_params=pltpu.CompilerParams(use_tc_tiling_on_sc=False),
        scratch_types=[
            pltpu.VMEM_SHARED((NUM_DST, CB), jnp.float32),
            pltpu.VMEM((NBUF, K, BLK), jnp.int32),
            pltpu.VMEM((NBUF, K, BLK), jnp.int32),
            pltpu.VMEM((NBUF, K, BLK), jnp.float32),
            pltpu.VMEM((K, BLK, CB), jnp.float32),
            pltpu.VMEM((ZR, CB), jnp.float32),
            pltpu.SemaphoreType.DMA,
            pltpu.SemaphoreType.DMA,
            pltpu.SemaphoreType.DMA,
        ],
    )
    outg = f(xg, colr, rowr, valr)
    return (outg.reshape(NCHUNK, NUM_DST, CB)
            .transpose(0, 2, 1)
            .reshape(BATCH, NUM_DST))
